# Initial kernel scaffold; baseline (speedup 1.0000x reference)
import jax
import jax.numpy as jnp
from jax.experimental import pallas as pl


def kernel(x, edge_index, batch, embed, W1, b1, W2, b2):
    raise NotImplementedError("write your pallas kernel here")


# speedup vs baseline: 11.9715x; 11.9715x over previous
import functools

import jax
import jax.numpy as jnp
from jax import lax
from jax.experimental import pallas as pl
from jax.experimental.pallas import tpu as pltpu
from jax.experimental.pallas import tpu_sc as plsc

N = 10000
E = 320000
H = 128
G = 512
NP = 10240

NC = 2
NS = 16
NW = NC * NS

_MESH = plsc.VectorSubcoreMesh(core_axis_name="c", subcore_axis_name="s")


def _make_conv():
    EW = E // NW
    C = 80
    NCHUNK = EW // C
    RPT = NP // NS

    @functools.partial(
        pl.kernel,
        out_type=jax.ShapeDtypeStruct((NC, NP, H), jnp.float32),
        mesh=_MESH,
        scratch_types=[
            pltpu.VMEM((C,), jnp.int32),
            pltpu.VMEM((C,), jnp.int32),
            pltpu.VMEM((C, H), jnp.float32),
            pltpu.VMEM_SHARED((NP, H), jnp.float32),
            pltpu.SemaphoreType.DMA,
        ],
    )
    def conv(u_hbm, src_hbm, dst_hbm, out_hbm, idx_s, idx_d, rows, acc, sem):
        c = lax.axis_index("c")
        s = lax.axis_index("s")
        wid = s * NC + c
        r0 = s * RPT
        pltpu.sync_copy(u_hbm.at[pl.ds(r0, RPT)], acc.at[pl.ds(r0, RPT)])
        plsc.subcore_barrier()
        base = wid * EW

        def body(j, carry):
            off = base + j * C
            pltpu.sync_copy(src_hbm.at[pl.ds(off, C)], idx_s)
            pltpu.sync_copy(dst_hbm.at[pl.ds(off, C)], idx_d)
            pltpu.async_copy(u_hbm.at[idx_s], rows, sem).wait()
            pltpu.sync_copy(rows, acc.at[idx_d], add=True)
            return carry

        lax.fori_loop(0, NCHUNK, body, 0)
        plsc.subcore_barrier()
        pltpu.sync_copy(acc.at[pl.ds(r0, RPT)], out_hbm.at[c, pl.ds(r0, RPT)])

    return conv


def _make_deg():
    EW = E // NW
    C = 80
    NCHUNK = EW // C
    RPT = NP // NS
    DW = 16

    @functools.partial(
        pl.kernel,
        out_type=jax.ShapeDtypeStruct((NC, NP, DW), jnp.float32),
        mesh=_MESH,
        scratch_types=[
            pltpu.VMEM((C,), jnp.int32),
            pltpu.VMEM((C, DW), jnp.float32),
            pltpu.VMEM_SHARED((NP, DW), jnp.float32),
        ],
    )
    def deg(dst_hbm, ones_hbm, zeros_hbm, out_hbm, idx_d, ones_v, hist):
        c = lax.axis_index("c")
        s = lax.axis_index("s")
        wid = s * NC + c
        r0 = s * RPT
        pltpu.sync_copy(zeros_hbm.at[pl.ds(r0, RPT)], hist.at[pl.ds(r0, RPT)])
        pltpu.sync_copy(ones_hbm, ones_v)
        plsc.subcore_barrier()
        base = wid * EW

        def body(j, carry):
            off = base + j * C
            pltpu.sync_copy(dst_hbm.at[pl.ds(off, C)], idx_d)
            pltpu.sync_copy(ones_v, hist.at[idx_d], add=True)
            return carry

        lax.fori_loop(0, NCHUNK, body, 0)
        plsc.subcore_barrier()
        pltpu.sync_copy(hist.at[pl.ds(r0, RPT)], out_hbm.at[c, pl.ds(r0, RPT)])

    return deg


def _make_segsum():
    RPT = NP // NS
    C = 128
    NCHUNK = RPT // C
    GPT = G // NS

    @functools.partial(
        pl.kernel,
        out_type=jax.ShapeDtypeStruct((NC, G, H), jnp.float32),
        mesh=_MESH,
        scratch_types=[
            pltpu.VMEM((C,), jnp.int32),
            pltpu.VMEM((C, H), jnp.float32),
            pltpu.VMEM_SHARED((G, H), jnp.float32),
            pltpu.SemaphoreType.DMA,
        ],
    )
    def segsum(h_hbm, batch_hbm, zeros_hbm, out_hbm, idx_b, rows, acc, sem):
        c = lax.axis_index("c")
        s = lax.axis_index("s")
        g0 = s * GPT
        pltpu.sync_copy(zeros_hbm.at[pl.ds(g0, GPT)], acc.at[pl.ds(g0, GPT)])
        plsc.subcore_barrier()
        r0 = s * RPT

        def body(j, carry):
            off = r0 + j * C
            pltpu.sync_copy(batch_hbm.at[pl.ds(off, C)], idx_b)
            pltpu.async_copy(h_hbm.at[pl.ds(off, C)], rows, sem).wait()
            pltpu.sync_copy(rows, acc.at[idx_b], add=True)
            return carry

        lax.fori_loop(0, NCHUNK, body, 0)
        plsc.subcore_barrier()
        pltpu.sync_copy(acc.at[pl.ds(g0, GPT)], out_hbm.at[c, pl.ds(g0, GPT)])

    return segsum


_conv_call = _make_conv()
_deg_call = _make_deg()
_segsum_call = _make_segsum()


_BM = 1024
_NBLK = 10


def _tc1_body(x_ref, hist_ref, embed_ref, w1_ref, u1_ref, dinv_ref):
    xb = x_ref[...]
    deg = 1.0 + hist_ref[0, :, 0:1] + hist_ref[1, :, 0:1]
    dinv = jax.lax.rsqrt(deg)
    dinv_b = jnp.broadcast_to(dinv, (_BM, H))
    cls = lax.broadcasted_iota(jnp.int32, (1, 32), 1)
    onehot = (xb == cls).astype(jnp.float32)
    ew1 = jnp.dot(embed_ref[...], w1_ref[...],
                  preferred_element_type=jnp.float32)
    u1 = dinv_b * jnp.dot(onehot, ew1, preferred_element_type=jnp.float32)
    u1_ref[...] = u1
    dinv_ref[...] = dinv_b


def _tc1(x, hist, embed32, w1):
    return pl.pallas_call(
        _tc1_body,
        grid=(_NBLK,),
        in_specs=[
            pl.BlockSpec((_BM, 1), lambda i: (i, 0)),
            pl.BlockSpec((NC, _BM, 16), lambda i: (0, i, 0)),
            pl.BlockSpec((32, H), lambda i: (0, 0)),
            pl.BlockSpec((H, H), lambda i: (0, 0)),
        ],
        out_specs=[
            pl.BlockSpec((_BM, H), lambda i: (i, 0)),
            pl.BlockSpec((_BM, H), lambda i: (i, 0)),
        ],
        out_shape=[
            jax.ShapeDtypeStruct((NP, H), jnp.float32),
            jax.ShapeDtypeStruct((NP, H), jnp.float32),
        ],
    )(x, hist, embed32, w1)


def _tc2_body(s_ref, u1_ref, dinv_ref, b1_ref, w2_ref, u2_ref):
    dinv = dinv_ref[...]
    pre = dinv * (s_ref[0] + s_ref[1] - u1_ref[...]) + b1_ref[...]
    h1 = jnp.maximum(pre, 0.0)
    u2_ref[...] = dinv * jnp.dot(h1, w2_ref[...],
                                 preferred_element_type=jnp.float32)


def _tc2(s1, u1, dinv_b, b1r, w2):
    return pl.pallas_call(
        _tc2_body,
        grid=(_NBLK,),
        in_specs=[
            pl.BlockSpec((NC, _BM, H), lambda i: (0, i, 0)),
            pl.BlockSpec((_BM, H), lambda i: (i, 0)),
            pl.BlockSpec((_BM, H), lambda i: (i, 0)),
            pl.BlockSpec((1, H), lambda i: (0, 0)),
            pl.BlockSpec((H, H), lambda i: (0, 0)),
        ],
        out_specs=pl.BlockSpec((_BM, H), lambda i: (i, 0)),
        out_shape=jax.ShapeDtypeStruct((NP, H), jnp.float32),
    )(s1, u1, dinv_b, b1r, w2)


def _tc3_body(s_ref, u2_ref, dinv_ref, b2_ref, h2_ref):
    i = pl.program_id(0)
    row = i * _BM + lax.broadcasted_iota(jnp.int32, (_BM, 1), 0)
    valid = row < N
    h2 = dinv_ref[...] * (s_ref[0] + s_ref[1] - u2_ref[...]) + b2_ref[...]
    h2_ref[...] = jnp.where(valid, h2, 0.0)


def _tc3(s2, u2, dinv_b, b2r):
    return pl.pallas_call(
        _tc3_body,
        grid=(_NBLK,),
        in_specs=[
            pl.BlockSpec((NC, _BM, H), lambda i: (0, i, 0)),
            pl.BlockSpec((_BM, H), lambda i: (i, 0)),
            pl.BlockSpec((_BM, H), lambda i: (i, 0)),
            pl.BlockSpec((1, H), lambda i: (0, 0)),
        ],
        out_specs=pl.BlockSpec((_BM, H), lambda i: (i, 0)),
        out_shape=jax.ShapeDtypeStruct((NP, H), jnp.float32),
    )(s2, u2, dinv_b, b2r)


@jax.jit
def kernel(x, edge_index, batch, embed, W1, b1, W2, b2):
    src = edge_index[0]
    dst = edge_index[1]
    embed32 = jnp.zeros((32, H), embed.dtype).at[:28].set(embed)
    ones16 = jnp.ones((80, 16), jnp.float32)
    zeros16 = jnp.zeros((NP, 16), jnp.float32)
    zeros_g = jnp.zeros((G, H), jnp.float32)
    batch_pad = jnp.zeros((NP,), jnp.int32).at[:N].set(batch)

    hist = _deg_call(dst, ones16, zeros16)
    u1, dinv_b = _tc1(x, hist, embed32, W1)
    s1 = _conv_call(u1, src, dst)
    u2 = _tc2(s1, u1, dinv_b, b1.reshape(1, H), W2)
    s2 = _conv_call(u2, src, dst)
    h2 = _tc3(s2, u2, dinv_b, b2.reshape(1, H))
    y = _segsum_call(h2, batch_pad, zeros_g)
    return (y[0], x)
